# baseline (speedup 1.0000x reference)
import functools

import jax
import jax.numpy as jnp
from jax import lax
from jax.experimental import pallas as pl
from jax.experimental.pallas import tpu as pltpu
from jax.experimental.pallas import tpu_sc as plsc

B = 16384
NC, NS = 2, 16
NW = NC * NS
BPW = B // NW
CH = 128
NCHUNK = BPW // CH
PH = 2
CPP = NCHUNK // PH


@functools.lru_cache(maxsize=None)
def _make_sc_gather():
    mesh = plsc.VectorSubcoreMesh(
        core_axis_name="c", subcore_axis_name="s",
        num_cores=NC, num_subcores=NS)

    @functools.partial(
        pl.kernel,
        out_type=(
            jax.ShapeDtypeStruct((B, 128), jnp.float32),
            jax.ShapeDtypeStruct((B, 128), jnp.float32),
            jax.ShapeDtypeStruct((B, 128), jnp.float32),
        ),
        mesh=mesh,
        scratch_types=(
            pltpu.VMEM((NCHUNK, CH), jnp.int32),
            pltpu.VMEM((NCHUNK, CH), jnp.int32),
            pltpu.VMEM((NCHUNK, CH), jnp.int32),
            pltpu.VMEM((CPP * CH, 128), jnp.float32),
            pltpu.VMEM((CPP * CH, 128), jnp.float32),
            pltpu.VMEM((CPP * CH, 128), jnp.float32),
            pltpu.SemaphoreType.DMA,
        ),
    )
    def _sc_gather(urow_hbm, mrow_hbm, rrow_hbm, ut_hbm, mt_hbm, rt_hbm,
                   u_out, m_out, r_out,
                   uidx, midx, ridx, urows, mrows, rrows, sem):
        wid = lax.axis_index("s") * NC + lax.axis_index("c")
        row0 = wid * NCHUNK

        pltpu.sync_copy(urow_hbm.at[pl.ds(row0, NCHUNK)], uidx)
        pltpu.sync_copy(mrow_hbm.at[pl.ds(row0, NCHUNK)], midx)
        pltpu.sync_copy(rrow_hbm.at[pl.ds(row0, NCHUNK)], ridx)

        base = wid * BPW
        for h in range(PH):
            copies = []
            for jj in range(CPP):
                j = h * CPP + jj
                copies.append(pltpu.async_copy(
                    ut_hbm.at[uidx.at[j]], urows.at[pl.ds(jj * CH, CH)], sem))
                copies.append(pltpu.async_copy(
                    mt_hbm.at[midx.at[j]], mrows.at[pl.ds(jj * CH, CH)], sem))
                copies.append(pltpu.async_copy(
                    rt_hbm.at[ridx.at[j]], rrows.at[pl.ds(jj * CH, CH)], sem))
            for c in copies:
                c.wait()
            off = base + h * CPP * CH
            pltpu.sync_copy(urows, u_out.at[pl.ds(off, CPP * CH)])
            pltpu.sync_copy(mrows, m_out.at[pl.ds(off, CPP * CH)])
            pltpu.sync_copy(rrows, r_out.at[pl.ds(off, CPP * CH)])

    return _sc_gather


BLK = 2048


def _select(w, pos, width):
    nsel = w.shape[1] // width
    out = w[:, 0:width]
    for p in range(1, nsel):
        out = jnp.where(pos == p, w[:, width * p:width * (p + 1)], out)
    return out


def _dense_body(u_ref, m_ref, r_ref, kw_ref, age_ref, upos_ref, mpos_ref,
                rpos_ref, kwW_ref, kwb_ref, W1u_ref, W1m_ref, W1r_ref,
                W1k_ref, w1a_ref, b1_ref, W2_ref, b2_ref, out_ref):
    u = _select(u_ref[...], upos_ref[...], 32)
    m = _select(m_ref[...], mpos_ref[...], 32)
    rpos = rpos_ref[...]
    r = _select(_select(r_ref[...], rpos >> 2, 32), rpos & 3, 8)
    k = jnp.maximum(
        jnp.dot(kw_ref[...], kwW_ref[...], preferred_element_type=jnp.float32)
        + kwb_ref[...], 0.0)
    acc = jnp.dot(u, W1u_ref[...], preferred_element_type=jnp.float32)
    acc += jnp.dot(m, W1m_ref[...], preferred_element_type=jnp.float32)
    acc += jnp.dot(r, W1r_ref[...], preferred_element_type=jnp.float32)
    acc += jnp.dot(k, W1k_ref[...], preferred_element_type=jnp.float32)
    acc += age_ref[...] * w1a_ref[...]
    h = jnp.maximum(acc + b1_ref[...], 0.0)
    o = jnp.dot(h, W2_ref[...], preferred_element_type=jnp.float32) + b2_ref[...]
    out_ref[...] = 1.0 / (1.0 + jnp.exp(-o))


def _dense(u, m, r, keywords, age2d, upos, mpos, rpos,
           kwW, kwb, W1u, W1m, W1r, W1k, w1a, b1, W2, b2):
    grid = (B // BLK,)
    blk = lambda w: pl.BlockSpec((BLK, w), lambda i: (i, 0))
    rep = lambda s0, s1: pl.BlockSpec((s0, s1), lambda i: (0, 0))
    return pl.pallas_call(
        _dense_body,
        grid=grid,
        in_specs=[
            blk(128), blk(128), blk(128), blk(64), blk(1), blk(1), blk(1),
            blk(1),
            rep(64, 16), rep(1, 16), rep(32, 64), rep(32, 64), rep(8, 64),
            rep(16, 64), rep(1, 64), rep(1, 64), rep(64, 1), rep(1, 1),
        ],
        out_specs=blk(1),
        out_shape=jax.ShapeDtypeStruct((B, 1), jnp.float32),
    )(u, m, r, keywords, age2d, upos, mpos, rpos,
      kwW, kwb, W1u, W1m, W1r, W1k, w1a, b1, W2, b2)


def kernel(user, movie, region, keywords, age, user_table, movie_table,
           region_table, kw_W, kw_b, W1, b1, W2, b2):
    rt128 = jnp.pad(region_table, ((0, 8), (0, 0))).reshape(-1, 128)
    u, m, r = _make_sc_gather()(
        (user >> 2).reshape(NW * NCHUNK, CH),
        (movie >> 2).reshape(NW * NCHUNK, CH),
        (region >> 4).reshape(NW * NCHUNK, CH),
        user_table.reshape(-1, 128), movie_table.reshape(-1, 128),
        rt128)
    out = _dense(
        u, m, r, keywords, age[:, None],
        (user & 3)[:, None], (movie & 3)[:, None], (region & 15)[:, None],
        kw_W, kw_b[None, :],
        W1[0:32], W1[32:64], W1[64:72], W1[72:88], W1[88:89],
        b1[None, :], W2, b2[None, :])
    return out[:, 0]


# speedup vs baseline: 1.7133x; 1.7133x over previous
import functools

import jax
import jax.numpy as jnp
from jax import lax
from jax.experimental import pallas as pl
from jax.experimental.pallas import tpu as pltpu
from jax.experimental.pallas import tpu_sc as plsc

B = 16384
NC, NS = 2, 16
NW = NC * NS
BPW = B // NW
CH = 128
NCHUNK = BPW // CH
PH = 2
P = BPW // PH


@functools.lru_cache(maxsize=None)
def _make_sc_gather():
    mesh = plsc.VectorSubcoreMesh(
        core_axis_name="c", subcore_axis_name="s",
        num_cores=NC, num_subcores=NS)

    @functools.partial(
        pl.kernel,
        out_type=(
            jax.ShapeDtypeStruct((B, 32), jnp.float32),
            jax.ShapeDtypeStruct((B, 32), jnp.float32),
            jax.ShapeDtypeStruct((B, 8), jnp.float32),
        ),
        mesh=mesh,
        scratch_types=(
            pltpu.VMEM((NCHUNK, CH), jnp.int32),
            pltpu.VMEM((NCHUNK, CH), jnp.int32),
            pltpu.VMEM((NCHUNK, CH), jnp.int32),
            pltpu.VMEM((P, 32), jnp.float32),
            pltpu.VMEM((P, 32), jnp.float32),
            pltpu.VMEM((P, 8), jnp.float32),
            pltpu.SemaphoreType.DMA,
        ),
    )
    def _sc_gather(user_hbm, movie_hbm, region_hbm, ut_hbm, mt_hbm, rt_hbm,
                   u_out, m_out, r_out,
                   uidx, midx, ridx, urows, mrows, rrows, sem):
        wid = lax.axis_index("s") * NC + lax.axis_index("c")
        row0 = wid * NCHUNK

        pltpu.sync_copy(user_hbm.at[pl.ds(row0, NCHUNK)], uidx)
        pltpu.sync_copy(movie_hbm.at[pl.ds(row0, NCHUNK)], midx)
        pltpu.sync_copy(region_hbm.at[pl.ds(row0, NCHUNK)], ridx)

        base = wid * BPW
        for h in range(PH):
            goff = h * (P // 16)

            def body(g, carry):
                i16 = g * 16
                j = (goff * 16 + i16) // CH
                k = (goff * 16 + i16) % CH
                uvec = uidx[j, pl.ds(k, 16)]
                mvec = midx[j, pl.ds(k, 16)]
                rvec = ridx[j, pl.ds(k, 16)]
                for l in range(16):
                    pltpu.async_copy(ut_hbm.at[uvec[l]], urows.at[i16 + l], sem)
                    pltpu.async_copy(mt_hbm.at[mvec[l]], mrows.at[i16 + l], sem)
                    pltpu.async_copy(rt_hbm.at[rvec[l]], rrows.at[i16 + l], sem)
                return carry

            lax.fori_loop(0, P // 16, body, 0)

            off = base + h * P
            pltpu.make_async_copy(u_out.at[pl.ds(off, P)], urows, sem).wait()
            pltpu.make_async_copy(m_out.at[pl.ds(off, P)], mrows, sem).wait()
            pltpu.make_async_copy(r_out.at[pl.ds(off, P)], rrows, sem).wait()

            pltpu.sync_copy(urows, u_out.at[pl.ds(off, P)])
            pltpu.sync_copy(mrows, m_out.at[pl.ds(off, P)])
            pltpu.sync_copy(rrows, r_out.at[pl.ds(off, P)])

    return _sc_gather


BLK = 2048


def _dense_body(u_ref, m_ref, r_ref, kw_ref, age_ref,
                kwW_ref, kwb_ref, W1u_ref, W1m_ref, W1r_ref, W1k_ref,
                w1a_ref, b1_ref, W2_ref, b2_ref, out_ref):
    k = jnp.maximum(
        jnp.dot(kw_ref[...], kwW_ref[...], preferred_element_type=jnp.float32)
        + kwb_ref[...], 0.0)
    acc = jnp.dot(u_ref[...], W1u_ref[...], preferred_element_type=jnp.float32)
    acc += jnp.dot(m_ref[...], W1m_ref[...], preferred_element_type=jnp.float32)
    acc += jnp.dot(r_ref[...], W1r_ref[...], preferred_element_type=jnp.float32)
    acc += jnp.dot(k, W1k_ref[...], preferred_element_type=jnp.float32)
    acc += age_ref[...] * w1a_ref[...]
    h = jnp.maximum(acc + b1_ref[...], 0.0)
    o = jnp.dot(h, W2_ref[...], preferred_element_type=jnp.float32) + b2_ref[...]
    out_ref[...] = 1.0 / (1.0 + jnp.exp(-o))


def _dense(u, m, r, keywords, age2d, kwW, kwb, W1u, W1m, W1r, W1k, w1a, b1,
           W2, b2):
    grid = (B // BLK,)
    blk = lambda w: pl.BlockSpec((BLK, w), lambda i: (i, 0))
    rep = lambda s0, s1: pl.BlockSpec((s0, s1), lambda i: (0, 0))
    return pl.pallas_call(
        _dense_body,
        grid=grid,
        in_specs=[
            blk(32), blk(32), blk(8), blk(64), blk(1),
            rep(64, 16), rep(1, 16), rep(32, 64), rep(32, 64), rep(8, 64),
            rep(16, 64), rep(1, 64), rep(1, 64), rep(64, 1), rep(1, 1),
        ],
        out_specs=blk(1),
        out_shape=jax.ShapeDtypeStruct((B, 1), jnp.float32),
    )(u, m, r, keywords, age2d, kwW, kwb, W1u, W1m, W1r, W1k, w1a, b1, W2, b2)


def kernel(user, movie, region, keywords, age, user_table, movie_table,
           region_table, kw_W, kw_b, W1, b1, W2, b2):
    u, m, r = _make_sc_gather()(
        user.reshape(NW * NCHUNK, CH),
        movie.reshape(NW * NCHUNK, CH),
        region.reshape(NW * NCHUNK, CH),
        user_table, movie_table, region_table)
    out = _dense(
        u, m, r, keywords, age[:, None],
        kw_W, kw_b[None, :],
        W1[0:32], W1[32:64], W1[64:72], W1[72:88], W1[88:89],
        b1[None, :], W2, b2[None, :])
    return out[:, 0]
